# Initial kernel scaffold; baseline (speedup 1.0000x reference)
import functools

import jax
import jax.numpy as jnp
from jax import lax
from jax.experimental import pallas as pl
from jax.experimental.pallas import tpu as pltpu
from jax.experimental.pallas import tpu_sc as plsc

_D = 128
_H = 8
_HD = 16
_ROW = 144
_NTILES = 32


def _proj_body(x_ref, w_ref, b_ref, o_ref):
    o_ref[...] = (
        jnp.dot(x_ref[...], w_ref[...], preferred_element_type=jnp.float32)
        + b_ref[...]
    )


def _project(x, w, b, blk):
    n, d_in = x.shape
    d_out = w.shape[1]
    return pl.pallas_call(
        _proj_body,
        grid=(n // blk,),
        in_specs=[
            pl.BlockSpec((blk, d_in), lambda i: (i, 0)),
            pl.BlockSpec((d_in, d_out), lambda i: (0, 0)),
            pl.BlockSpec((1, d_out), lambda i: (0, 0)),
        ],
        out_specs=pl.BlockSpec((blk, d_out), lambda i: (i, 0)),
        out_shape=jax.ShapeDtypeStruct((n, d_out), jnp.float32),
    )(x, w, b.reshape(1, d_out))


def _make_sc_edge_kernel(E, N, C):
    ept = E // _NTILES
    n_chunks = ept // C
    rpt = N // 16
    mesh = plsc.VectorSubcoreMesh(core_axis_name="c", subcore_axis_name="s")

    @functools.partial(
        pl.kernel,
        out_type=jax.ShapeDtypeStruct((2, N, _ROW), jnp.float32),
        mesh=mesh,
        scratch_types=[
            pltpu.VMEM((C,), jnp.int32),
            pltpu.VMEM((C,), jnp.int32),
            pltpu.VMEM((C, _D), jnp.float32),
            pltpu.VMEM((C, _D), jnp.float32),
            pltpu.VMEM((C, _D), jnp.float32),
            pltpu.VMEM((C, _ROW), jnp.float32),
            pltpu.VMEM((_H, _HD), jnp.float32),
            pltpu.VMEM_SHARED((N, _ROW), jnp.float32),
            pltpu.SemaphoreType.DMA,
            pltpu.SemaphoreType.DMA,
        ],
    )
    def k(h_hbm, ep_hbm, snd_hbm, rcv_hbm, a_hbm, z_hbm, out_hbm,
          snd_v, rcv_v, hs_v, hr_v, ep_v, md_v, a_v, acc, sem1, sem2):
        cid = lax.axis_index("c")
        sid = lax.axis_index("s")
        wid = cid * 16 + sid
        r0 = sid * rpt
        pltpu.sync_copy(z_hbm.at[pl.ds(r0, rpt)], acc.at[pl.ds(r0, rpt)])
        pltpu.sync_copy(a_hbm, a_v)
        plsc.subcore_barrier()

        base = wid * ept
        lane = lax.iota(jnp.int32, 16)

        def chunk_body(i, carry):
            off = base + i * C
            pltpu.sync_copy(snd_hbm.at[pl.ds(off, C)], snd_v)
            pltpu.sync_copy(rcv_hbm.at[pl.ds(off, C)], rcv_v)
            cp1 = pltpu.async_copy(h_hbm.at[snd_v], hs_v, sem1)
            cp2 = pltpu.async_copy(h_hbm.at[rcv_v], hr_v, sem2)
            pltpu.sync_copy(ep_hbm.at[pl.ds(off, C)], ep_v)
            cp1.wait()
            cp2.wait()

            def edge_body(e, carry2):
                hs = []
                m = []
                for j in range(_H):
                    hs_j = hs_v[e, pl.ds(j * _HD, _HD)]
                    x_j = (hs_j + hr_v[e, pl.ds(j * _HD, _HD)]
                           + ep_v[e, pl.ds(j * _HD, _HD)])
                    t = jnp.exp(jnp.minimum(x_j, 20.0))
                    u = t + 1.0
                    sq = u * u
                    m_j = x_j * ((sq - 1.0) / (sq + 1.0))
                    hs.append(hs_j)
                    m.append(m_j)
                lvec = jnp.zeros(16, jnp.float32)
                for j in range(_H):
                    l_j = jnp.sum(m[j] * a_v[j, :])
                    lvec = jnp.where(lane == j, l_j, lvec)
                w = jnp.exp(jnp.minimum(lvec, 50.0))
                for j in range(_H):
                    md_v[e, pl.ds(j * _HD, _HD)] = w[j] * hs[j]
                md_v[e, pl.ds(_D, _HD)] = w
                return carry2

            lax.fori_loop(0, C, edge_body, 0)
            pltpu.sync_copy(md_v, acc.at[rcv_v], add=True)
            return carry

        lax.fori_loop(0, n_chunks, chunk_body, 0)
        plsc.subcore_barrier()
        pltpu.sync_copy(acc.at[pl.ds(r0, rpt)],
                        out_hbm.at[cid, pl.ds(r0, rpt)])

    return k


def _combine_body(p_ref, o_ref):
    s = p_ref[0] + p_ref[1]
    msg = s[:, :_D]
    den = s[:, _D:_D + _H]
    col = lax.broadcasted_iota(jnp.int32, (_H, _D), 1) // _HD
    row = lax.broadcasted_iota(jnp.int32, (_H, _D), 0)
    expand = (col == row).astype(jnp.float32)
    denb = jnp.dot(den, expand, preferred_element_type=jnp.float32)
    o_ref[...] = jnp.where(denb > 0.0, msg / denb, 0.0)


def _combine(parts, N, blk):
    return pl.pallas_call(
        _combine_body,
        grid=(N // blk,),
        in_specs=[pl.BlockSpec((2, blk, _ROW), lambda i: (0, i, 0))],
        out_specs=pl.BlockSpec((blk, _D), lambda i: (i, 0)),
        out_shape=jax.ShapeDtypeStruct((N, _D), jnp.float32),
    )(parts)


def kernel(node_features, senders, receivers, edge_features, W_kernel,
           W_bias, We_kernel, We_bias, a_kernel):
    N = node_features.shape[0]
    E = senders.shape[0]
    h = _project(node_features, W_kernel, W_bias, 1000)
    ep = _project(edge_features, We_kernel, We_bias, 2000)
    zeros = jnp.zeros((N, _ROW), jnp.float32)
    sc = _make_sc_edge_kernel(E, N, 80)
    parts = sc(h, ep, senders.astype(jnp.int32), receivers.astype(jnp.int32),
               a_kernel.astype(jnp.float32), zeros)
    return _combine(parts, N, 1000)


# speedup vs baseline: 1.3608x; 1.3608x over previous
import functools

import jax
import jax.numpy as jnp
from jax import lax
from jax.experimental import pallas as pl
from jax.experimental.pallas import tpu as pltpu
from jax.experimental.pallas import tpu_sc as plsc

_D = 128
_H = 8
_HD = 16
_ROW = 144
_NTILES = 32


def _proj_body(x_ref, w_ref, b_ref, o_ref):
    o_ref[...] = (
        jnp.dot(x_ref[...], w_ref[...], preferred_element_type=jnp.float32)
        + b_ref[...]
    )


def _project(x, w, b, blk):
    n, d_in = x.shape
    d_out = w.shape[1]
    return pl.pallas_call(
        _proj_body,
        grid=(n // blk,),
        in_specs=[
            pl.BlockSpec((blk, d_in), lambda i: (i, 0)),
            pl.BlockSpec((d_in, d_out), lambda i: (0, 0)),
            pl.BlockSpec((1, d_out), lambda i: (0, 0)),
        ],
        out_specs=pl.BlockSpec((blk, d_out), lambda i: (i, 0)),
        out_shape=jax.ShapeDtypeStruct((n, d_out), jnp.float32),
    )(x, w, b.reshape(1, d_out))


def _make_sc_edge_kernel(E, N, C):
    ept = E // _NTILES
    n_chunks = ept // C
    npad = -(-N // 16) * 16
    rpt = npad // 16
    mesh = plsc.VectorSubcoreMesh(core_axis_name="c", subcore_axis_name="s")

    @functools.partial(
        pl.kernel,
        out_type=jax.ShapeDtypeStruct((2, npad, _ROW), jnp.float32),
        mesh=mesh,
        compiler_params=pltpu.CompilerParams(
            needs_layout_passes=False, use_tc_tiling_on_sc=False),
        scratch_types=[
            pltpu.VMEM((C,), jnp.int32),
            pltpu.VMEM((C,), jnp.int32),
            pltpu.VMEM((C, _D), jnp.float32),
            pltpu.VMEM((C, _D), jnp.float32),
            pltpu.VMEM((C, _D), jnp.float32),
            pltpu.VMEM((C, _ROW), jnp.float32),
            pltpu.VMEM((_H, _HD), jnp.float32),
            pltpu.VMEM_SHARED((npad, _ROW), jnp.float32),
            pltpu.SemaphoreType.DMA,
            pltpu.SemaphoreType.DMA,
        ],
    )
    def k(h_hbm, ep_hbm, snd_hbm, rcv_hbm, a_hbm, z_hbm, out_hbm,
          snd_v, rcv_v, hs_v, hr_v, ep_v, md_v, a_v, acc, sem1, sem2):
        cid = lax.axis_index("c")
        sid = lax.axis_index("s")
        wid = cid * 16 + sid
        r0 = sid * rpt
        pltpu.sync_copy(z_hbm.at[pl.ds(r0, rpt)], acc.at[pl.ds(r0, rpt)])
        pltpu.sync_copy(a_hbm, a_v)
        plsc.subcore_barrier()

        base = wid * ept
        lane = lax.iota(jnp.int32, 16)

        def chunk_body(i, carry):
            off = base + i * C
            pltpu.sync_copy(snd_hbm.at[pl.ds(off, C)], snd_v)
            pltpu.sync_copy(rcv_hbm.at[pl.ds(off, C)], rcv_v)
            cp1 = pltpu.async_copy(h_hbm.at[snd_v], hs_v, sem1)
            cp2 = pltpu.async_copy(h_hbm.at[rcv_v], hr_v, sem2)
            pltpu.sync_copy(ep_hbm.at[pl.ds(off, C)], ep_v)
            cp1.wait()
            cp2.wait()

            def edge_body(e, carry2):
                wvec = jnp.zeros(16, jnp.float32)
                for j in range(_H):
                    hs_j = hs_v[e, pl.ds(j * _HD, _HD)]
                    x_j = (hs_j + hr_v[e, pl.ds(j * _HD, _HD)]
                           + ep_v[e, pl.ds(j * _HD, _HD)])
                    t = jnp.exp(jnp.minimum(x_j, 20.0))
                    u = t + 1.0
                    sq = u * u
                    m_j = x_j * ((sq - 1.0) / (sq + 1.0))
                    l_j = jnp.sum(m_j * a_v[j, :])
                    w_vj = jnp.exp(jnp.minimum(l_j + wvec * 0.0, 60.0))
                    md_v[e, pl.ds(j * _HD, _HD)] = w_vj * hs_j
                    wvec = jnp.where(lane == j, w_vj, wvec)
                md_v[e, pl.ds(_D, _HD)] = wvec
                return carry2

            lax.fori_loop(0, C, edge_body, 0)
            pltpu.sync_copy(md_v, acc.at[rcv_v], add=True)
            return carry

        lax.fori_loop(0, n_chunks, chunk_body, 0)
        plsc.subcore_barrier()
        pltpu.sync_copy(acc.at[pl.ds(r0, rpt)],
                        out_hbm.at[cid, pl.ds(r0, rpt)])

    return k


def _combine_body(p_ref, o_ref):
    s = p_ref[0] + p_ref[1]
    msg = s[:, :_D]
    den = s[:, _D:_D + _H]
    col = lax.broadcasted_iota(jnp.int32, (_H, _D), 1) // _HD
    row = lax.broadcasted_iota(jnp.int32, (_H, _D), 0)
    expand = (col == row).astype(jnp.float32)
    denb = jnp.dot(den, expand, preferred_element_type=jnp.float32)
    o_ref[...] = jnp.where(denb > 0.0, msg / denb, 0.0)


def _combine(parts, N, blk):
    return pl.pallas_call(
        _combine_body,
        grid=(N // blk,),
        in_specs=[pl.BlockSpec((2, blk, _ROW), lambda i: (0, i, 0))],
        out_specs=pl.BlockSpec((blk, _D), lambda i: (i, 0)),
        out_shape=jax.ShapeDtypeStruct((N, _D), jnp.float32),
    )(parts)


def kernel(node_features, senders, receivers, edge_features, W_kernel,
           W_bias, We_kernel, We_bias, a_kernel):
    N = node_features.shape[0]
    E = senders.shape[0]
    h = _project(node_features, W_kernel, W_bias, 1000)
    ep = _project(edge_features, We_kernel, We_bias, 2000)
    zeros = jnp.zeros((-(-N // 16) * 16, _ROW), jnp.float32)
    sc = _make_sc_edge_kernel(E, N, 40)
    parts = sc(h, ep, senders.astype(jnp.int32), receivers.astype(jnp.int32),
               a_kernel.astype(jnp.float32), zeros)
    return _combine(parts, N, 1000)


# speedup vs baseline: 4.9140x; 3.6110x over previous
import functools

import jax
import jax.numpy as jnp
from jax import lax
from jax.experimental import pallas as pl
from jax.experimental.pallas import tpu as pltpu
from jax.experimental.pallas import tpu_sc as plsc

_D = 128
_H = 8
_HD = 16
_ROW = 144
_NTILES = 32


def _proj_body(x_ref, w_ref, b_ref, o_ref):
    o_ref[...] = (
        jnp.dot(x_ref[...], w_ref[...], preferred_element_type=jnp.float32)
        + b_ref[...]
    )


def _project(x, w, b, blk):
    n, d_in = x.shape
    d_out = w.shape[1]
    return pl.pallas_call(
        _proj_body,
        grid=(n // blk,),
        in_specs=[
            pl.BlockSpec((blk, d_in), lambda i: (i, 0)),
            pl.BlockSpec((d_in, d_out), lambda i: (0, 0)),
            pl.BlockSpec((1, d_out), lambda i: (0, 0)),
        ],
        out_specs=pl.BlockSpec((blk, d_out), lambda i: (i, 0)),
        out_shape=jax.ShapeDtypeStruct((n, d_out), jnp.float32),
    )(x, w, b.reshape(1, d_out))


def _make_sc_gather_kernel(E, C):
    ept = E // _NTILES
    n_chunks = ept // C
    mesh = plsc.VectorSubcoreMesh(core_axis_name="c", subcore_axis_name="s")

    @functools.partial(
        pl.kernel,
        out_type=(
            jax.ShapeDtypeStruct((E, _D), jnp.float32),
            jax.ShapeDtypeStruct((E, _D), jnp.float32),
        ),
        mesh=mesh,
        compiler_params=pltpu.CompilerParams(
            needs_layout_passes=False, use_tc_tiling_on_sc=False),
        scratch_types=[
            pltpu.VMEM((C,), jnp.int32),
            pltpu.VMEM((C,), jnp.int32),
            pltpu.VMEM((C, _D), jnp.float32),
            pltpu.VMEM((C, _D), jnp.float32),
            pltpu.SemaphoreType.DMA,
            pltpu.SemaphoreType.DMA,
        ],
    )
    def k(h_hbm, snd_hbm, rcv_hbm, hs_out, hr_out,
          snd_v, rcv_v, hs_v, hr_v, sem1, sem2):
        cid = lax.axis_index("c")
        sid = lax.axis_index("s")
        base = (cid * 16 + sid) * ept

        def chunk_body(i, carry):
            off = base + i * C
            pltpu.sync_copy(snd_hbm.at[pl.ds(off, C)], snd_v)
            pltpu.sync_copy(rcv_hbm.at[pl.ds(off, C)], rcv_v)
            cp1 = pltpu.async_copy(h_hbm.at[snd_v], hs_v, sem1)
            cp2 = pltpu.async_copy(h_hbm.at[rcv_v], hr_v, sem2)
            cp1.wait()
            cp2.wait()
            pltpu.sync_copy(hs_v, hs_out.at[pl.ds(off, C)])
            pltpu.sync_copy(hr_v, hr_out.at[pl.ds(off, C)])
            return carry

        lax.fori_loop(0, n_chunks, chunk_body, 0)

    return k


def _edge_body(hs_ref, hr_ref, ef_ref, we_ref, be_ref, ad_ref, o_ref):
    hs = hs_ref[...]
    x = (hs + hr_ref[...] + be_ref[...]
         + jnp.dot(ef_ref[...], we_ref[...],
                   preferred_element_type=jnp.float32))
    t = jnp.exp(jnp.minimum(x, 20.0))
    u = t + 1.0
    sq = u * u
    m = x * ((sq - 1.0) / (sq + 1.0))
    logits = jnp.dot(m, ad_ref[...], preferred_element_type=jnp.float32)
    w = jnp.exp(jnp.minimum(logits, 60.0))
    col = lax.broadcasted_iota(jnp.int32, (_H, _D), 1) // _HD
    row = lax.broadcasted_iota(jnp.int32, (_H, _D), 0)
    expand = (col == row).astype(jnp.float32)
    w128 = jnp.dot(w, expand, preferred_element_type=jnp.float32)
    blk = hs.shape[0]
    o_ref[...] = jnp.concatenate(
        [w128 * hs, w, jnp.zeros((blk, _ROW - _D - _H), jnp.float32)], axis=1)


def _edge_compute(hs, hr, ef, we, be, ad, blk):
    E, de = ef.shape
    return pl.pallas_call(
        _edge_body,
        grid=(E // blk,),
        in_specs=[
            pl.BlockSpec((blk, _D), lambda i: (i, 0)),
            pl.BlockSpec((blk, _D), lambda i: (i, 0)),
            pl.BlockSpec((blk, de), lambda i: (i, 0)),
            pl.BlockSpec((de, _D), lambda i: (0, 0)),
            pl.BlockSpec((1, _D), lambda i: (0, 0)),
            pl.BlockSpec((_D, _H), lambda i: (0, 0)),
        ],
        out_specs=pl.BlockSpec((blk, _ROW), lambda i: (i, 0)),
        out_shape=jax.ShapeDtypeStruct((E, _ROW), jnp.float32),
    )(hs, hr, ef, we, be.reshape(1, _D), ad)


def _make_sc_scatter_kernel(E, N, C):
    ept = E // _NTILES
    n_chunks = ept // C
    npad = -(-N // 16) * 16
    rpt = npad // 16
    mesh = plsc.VectorSubcoreMesh(core_axis_name="c", subcore_axis_name="s")

    @functools.partial(
        pl.kernel,
        out_type=jax.ShapeDtypeStruct((2, npad, _ROW), jnp.float32),
        mesh=mesh,
        compiler_params=pltpu.CompilerParams(
            needs_layout_passes=False, use_tc_tiling_on_sc=False),
        scratch_types=[
            pltpu.VMEM((C,), jnp.int32),
            pltpu.VMEM((C, _ROW), jnp.float32),
            pltpu.VMEM_SHARED((npad, _ROW), jnp.float32),
        ],
    )
    def k(md_hbm, rcv_hbm, z_hbm, out_hbm, rcv_v, md_v, acc):
        cid = lax.axis_index("c")
        sid = lax.axis_index("s")
        r0 = sid * rpt
        pltpu.sync_copy(z_hbm.at[pl.ds(r0, rpt)], acc.at[pl.ds(r0, rpt)])
        plsc.subcore_barrier()

        base = (cid * 16 + sid) * ept

        def chunk_body(i, carry):
            off = base + i * C
            pltpu.sync_copy(rcv_hbm.at[pl.ds(off, C)], rcv_v)
            pltpu.sync_copy(md_hbm.at[pl.ds(off, C)], md_v)
            pltpu.sync_copy(md_v, acc.at[rcv_v], add=True)
            return carry

        lax.fori_loop(0, n_chunks, chunk_body, 0)
        plsc.subcore_barrier()
        pltpu.sync_copy(acc.at[pl.ds(r0, rpt)],
                        out_hbm.at[cid, pl.ds(r0, rpt)])

    return k


def _combine_body(p_ref, o_ref):
    s = p_ref[0] + p_ref[1]
    msg = s[:, :_D]
    den = s[:, _D:_D + _H]
    col = lax.broadcasted_iota(jnp.int32, (_H, _D), 1) // _HD
    row = lax.broadcasted_iota(jnp.int32, (_H, _D), 0)
    expand = (col == row).astype(jnp.float32)
    denb = jnp.dot(den, expand, preferred_element_type=jnp.float32)
    o_ref[...] = jnp.where(denb > 0.0, msg / denb, 0.0)


def _combine(parts, N, blk):
    return pl.pallas_call(
        _combine_body,
        grid=(N // blk,),
        in_specs=[pl.BlockSpec((2, blk, _ROW), lambda i: (0, i, 0))],
        out_specs=pl.BlockSpec((blk, _D), lambda i: (i, 0)),
        out_shape=jax.ShapeDtypeStruct((N, _D), jnp.float32),
    )(parts)


def kernel(node_features, senders, receivers, edge_features, W_kernel,
           W_bias, We_kernel, We_bias, a_kernel):
    N = node_features.shape[0]
    E = senders.shape[0]
    snd = senders.astype(jnp.int32)
    rcv = receivers.astype(jnp.int32)
    h = _project(node_features, W_kernel, W_bias, 1000)
    hs, hr = _make_sc_gather_kernel(E, 400)(h, snd, rcv)
    ad = (jnp.repeat(jnp.eye(_H, dtype=jnp.float32), _HD, axis=0)
          * a_kernel.reshape(_D, 1))
    md = _edge_compute(hs, hr, edge_features, We_kernel, We_bias, ad, 2000)
    zeros = jnp.zeros((-(-N // 16) * 16, _ROW), jnp.float32)
    parts = _make_sc_scatter_kernel(E, N, 200)(md, rcv, zeros)
    return _combine(parts, N, 1000)


# speedup vs baseline: 5.9542x; 1.2117x over previous
import functools

import jax
import jax.numpy as jnp
from jax import lax
from jax.experimental import pallas as pl
from jax.experimental.pallas import tpu as pltpu
from jax.experimental.pallas import tpu_sc as plsc

_D = 128
_H = 8
_HD = 16
_ROW = 144
_NTILES = 32


def _proj_body(x_ref, w_ref, b_ref, o_ref):
    o_ref[...] = (
        jnp.dot(x_ref[...], w_ref[...], preferred_element_type=jnp.float32)
        + b_ref[...]
    )


def _project(x, w, b, blk):
    n, d_in = x.shape
    d_out = w.shape[1]
    return pl.pallas_call(
        _proj_body,
        grid=(n // blk,),
        in_specs=[
            pl.BlockSpec((blk, d_in), lambda i: (i, 0)),
            pl.BlockSpec((d_in, d_out), lambda i: (0, 0)),
            pl.BlockSpec((1, d_out), lambda i: (0, 0)),
        ],
        out_specs=pl.BlockSpec((blk, d_out), lambda i: (i, 0)),
        out_shape=jax.ShapeDtypeStruct((n, d_out), jnp.float32),
    )(x, w, b.reshape(1, d_out))


def _make_sc_gather_kernel(E, C):
    ept = E // _NTILES
    n_chunks = ept // C
    mesh = plsc.VectorSubcoreMesh(core_axis_name="c", subcore_axis_name="s")

    @functools.partial(
        pl.kernel,
        out_type=(
            jax.ShapeDtypeStruct((E, _D), jnp.float32),
            jax.ShapeDtypeStruct((E, _D), jnp.float32),
        ),
        mesh=mesh,
        compiler_params=pltpu.CompilerParams(
            needs_layout_passes=False, use_tc_tiling_on_sc=False),
        scratch_types=[
            pltpu.VMEM((C,), jnp.int32),
            pltpu.VMEM((C,), jnp.int32),
            pltpu.VMEM((C, _D), jnp.float32),
            pltpu.VMEM((C, _D), jnp.float32),
            pltpu.SemaphoreType.DMA,
            pltpu.SemaphoreType.DMA,
        ],
    )
    def k(h_hbm, snd_hbm, rcv_hbm, hs_out, hr_out,
          snd_v, rcv_v, hs_v, hr_v, sem1, sem2):
        cid = lax.axis_index("c")
        sid = lax.axis_index("s")
        base = (cid * 16 + sid) * ept

        def chunk_body(i, carry):
            off = base + i * C
            pltpu.sync_copy(snd_hbm.at[pl.ds(off, C)], snd_v)
            pltpu.sync_copy(rcv_hbm.at[pl.ds(off, C)], rcv_v)
            cp1 = pltpu.async_copy(h_hbm.at[snd_v], hs_v, sem1)
            cp2 = pltpu.async_copy(h_hbm.at[rcv_v], hr_v, sem2)
            cp1.wait()
            cp2.wait()
            pltpu.sync_copy(hs_v, hs_out.at[pl.ds(off, C)])
            pltpu.sync_copy(hr_v, hr_out.at[pl.ds(off, C)])
            return carry

        lax.fori_loop(0, n_chunks, chunk_body, 0)

    return k


def _edge_body(hs_ref, hr_ref, ef_ref, we_ref, be_ref, ad_ref, o_ref, ow_ref):
    hs = hs_ref[...]
    x = (hs + hr_ref[...] + be_ref[...]
         + jnp.dot(ef_ref[...], we_ref[...],
                   preferred_element_type=jnp.float32))
    t = jnp.exp(jnp.minimum(x, 20.0))
    u = t + 1.0
    sq = u * u
    m = x * ((sq - 1.0) / (sq + 1.0))
    logits = jnp.dot(m, ad_ref[...], preferred_element_type=jnp.float32)
    w = jnp.exp(jnp.minimum(logits, 60.0))
    col = lax.broadcasted_iota(jnp.int32, (_H, _D), 1) // _HD
    row = lax.broadcasted_iota(jnp.int32, (_H, _D), 0)
    expand = (col == row).astype(jnp.float32)
    w128 = jnp.dot(w, expand, preferred_element_type=jnp.float32)
    blk = hs.shape[0]
    o_ref[...] = w128 * hs
    ow_ref[...] = jnp.concatenate(
        [w, jnp.zeros((blk, _HD - _H), jnp.float32)], axis=1)


def _edge_compute(hs, hr, ef, we, be, ad, blk):
    E, de = ef.shape
    return pl.pallas_call(
        _edge_body,
        grid=(E // blk,),
        in_specs=[
            pl.BlockSpec((blk, _D), lambda i: (i, 0)),
            pl.BlockSpec((blk, _D), lambda i: (i, 0)),
            pl.BlockSpec((blk, de), lambda i: (i, 0)),
            pl.BlockSpec((de, _D), lambda i: (0, 0)),
            pl.BlockSpec((1, _D), lambda i: (0, 0)),
            pl.BlockSpec((_D, _H), lambda i: (0, 0)),
        ],
        out_specs=[
            pl.BlockSpec((blk, _D), lambda i: (i, 0)),
            pl.BlockSpec((blk, _HD), lambda i: (i, 0)),
        ],
        out_shape=[
            jax.ShapeDtypeStruct((E, _D), jnp.float32),
            jax.ShapeDtypeStruct((E, _HD), jnp.float32),
        ],
    )(hs, hr, ef, we, be.reshape(1, _D), ad)


def _make_sc_scatter_kernel(E, N, C):
    ept = E // _NTILES
    n_chunks = ept // C
    npad = -(-N // 16) * 16
    rpt = npad // 16
    mesh = plsc.VectorSubcoreMesh(core_axis_name="c", subcore_axis_name="s")

    @functools.partial(
        pl.kernel,
        out_type=(
            jax.ShapeDtypeStruct((2, npad, _D), jnp.float32),
            jax.ShapeDtypeStruct((2, npad, _HD), jnp.float32),
        ),
        mesh=mesh,
        compiler_params=pltpu.CompilerParams(
            needs_layout_passes=False, use_tc_tiling_on_sc=False),
        scratch_types=[
            pltpu.VMEM((C,), jnp.int32),
            pltpu.VMEM((C, _D), jnp.float32),
            pltpu.VMEM((C, _HD), jnp.float32),
            pltpu.VMEM_SHARED((npad, _D), jnp.float32),
            pltpu.VMEM_SHARED((npad, _HD), jnp.float32),
        ],
    )
    def k(msg_hbm, w_hbm, rcv_hbm, zm_hbm, zw_hbm, outm_hbm, outw_hbm,
          rcv_v, msg_v, w_v, accm, accw):
        cid = lax.axis_index("c")
        sid = lax.axis_index("s")
        r0 = sid * rpt
        pltpu.sync_copy(zm_hbm.at[pl.ds(r0, rpt)], accm.at[pl.ds(r0, rpt)])
        pltpu.sync_copy(zw_hbm.at[pl.ds(r0, rpt)], accw.at[pl.ds(r0, rpt)])
        plsc.subcore_barrier()

        base = (cid * 16 + sid) * ept

        def chunk_body(i, carry):
            off = base + i * C
            pltpu.sync_copy(rcv_hbm.at[pl.ds(off, C)], rcv_v)
            pltpu.sync_copy(msg_hbm.at[pl.ds(off, C)], msg_v)
            pltpu.sync_copy(w_hbm.at[pl.ds(off, C)], w_v)
            pltpu.sync_copy(msg_v, accm.at[rcv_v], add=True)
            pltpu.sync_copy(w_v, accw.at[rcv_v], add=True)
            return carry

        lax.fori_loop(0, n_chunks, chunk_body, 0)
        plsc.subcore_barrier()
        pltpu.sync_copy(accm.at[pl.ds(r0, rpt)],
                        outm_hbm.at[cid, pl.ds(r0, rpt)])
        pltpu.sync_copy(accw.at[pl.ds(r0, rpt)],
                        outw_hbm.at[cid, pl.ds(r0, rpt)])

    return k


def _combine_body(pm_ref, pw_ref, o_ref):
    msg = pm_ref[0] + pm_ref[1]
    sw = pw_ref[0] + pw_ref[1]
    den = sw[:, :_H]
    col = lax.broadcasted_iota(jnp.int32, (_H, _D), 1) // _HD
    row = lax.broadcasted_iota(jnp.int32, (_H, _D), 0)
    expand = (col == row).astype(jnp.float32)
    denb = jnp.dot(den, expand, preferred_element_type=jnp.float32)
    o_ref[...] = jnp.where(denb > 0.0, msg / denb, 0.0)


def _combine(parts_m, parts_w, N, blk):
    return pl.pallas_call(
        _combine_body,
        grid=(N // blk,),
        in_specs=[
            pl.BlockSpec((2, blk, _D), lambda i: (0, i, 0)),
            pl.BlockSpec((2, blk, _HD), lambda i: (0, i, 0)),
        ],
        out_specs=pl.BlockSpec((blk, _D), lambda i: (i, 0)),
        out_shape=jax.ShapeDtypeStruct((N, _D), jnp.float32),
    )(parts_m, parts_w)


def kernel(node_features, senders, receivers, edge_features, W_kernel,
           W_bias, We_kernel, We_bias, a_kernel):
    N = node_features.shape[0]
    E = senders.shape[0]
    snd = senders.astype(jnp.int32)
    rcv = receivers.astype(jnp.int32)
    h = _project(node_features, W_kernel, W_bias, 1000)
    hs, hr = _make_sc_gather_kernel(E, 400)(h, snd, rcv)
    ad = (jnp.repeat(jnp.eye(_H, dtype=jnp.float32), _HD, axis=0)
          * a_kernel.reshape(_D, 1))
    msg, w16 = _edge_compute(hs, hr, edge_features, We_kernel, We_bias, ad,
                             2000)
    npad = -(-N // 16) * 16
    zm = jnp.zeros((npad, _D), jnp.float32)
    zw = jnp.zeros((npad, _HD), jnp.float32)
    parts_m, parts_w = _make_sc_scatter_kernel(E, N, 200)(msg, w16, rcv, zm, zw)
    return _combine(parts_m, parts_w, N, 1000)
